# baseline (speedup 1.0000x reference)
import functools

import jax
import jax.numpy as jnp
from jax import lax
from jax.experimental import pallas as pl
from jax.experimental.pallas import tpu as pltpu
from jax.experimental.pallas import tpu_sc as plsc

N = 16384
D = 128
K = N // 2
MARGIN = 2.0
EPS = 1e-6

_L = 16
_NT = 16
_PW = N // _NT
_NV = _PW // _L
_RC = 64
_CW = _RC * D
_NCH = _PW // _RC


def _full_sc_body(o1_hbm, o2_hbm, lab_hbm, out_hbm,
                  a0_v, b0_v, a1_v, b1_v, lab_v, vals_v,
                  row_i_v, all_i_v, row_f_v, all_f_v, res_v,
                  cnts_sh, sums_sh,
                  sema0, semb0, sema1, semb1):
    wid = lax.axis_index("s")
    iota = lax.iota(jnp.int32, _L)
    base = wid * (_PW * D)

    pltpu.sync_copy(lab_hbm.at[pl.ds(wid * _PW, _PW)], lab_v)

    def compute_chunk(c, av, bv):
        def group_body(g, _):
            svec = jnp.zeros((_L,), jnp.float32)
            for i in range(_L):
                rowbase = (g * _L + i) * D
                acc = jnp.zeros((_L,), jnp.float32)
                for q in range(D // _L):
                    va = av[pl.ds(rowbase + q * _L, _L)]
                    vb = bv[pl.ds(rowbase + q * _L, _L)]
                    d = va - vb + EPS
                    acc = acc + d * d
                svec = jnp.where(iota == i, jnp.sum(acc), svec)
            sb = jnp.maximum(svec, 1e-30)
            ib = plsc.bitcast(sb, jnp.int32)
            r = plsc.bitcast(jnp.int32(0x5F3759DF) - lax.shift_right_logical(ib, 1),
                             jnp.float32)
            for _unused in range(3):
                r = r * (1.5 - 0.5 * sb * r * r)
            dist = sb * r
            labf = lab_v[pl.ds(c * _RC + g * _L, _L)].astype(jnp.float32)
            hin = jnp.maximum(MARGIN - dist, 0.0)
            vals_v[pl.ds(c * _RC + g * _L, _L)] = (
                labf * svec + (1.0 - labf) * hin * hin)
            return 0

        lax.fori_loop(0, _RC // _L, group_body, 0)

    pltpu.async_copy(o1_hbm.at[pl.ds(base, _CW)], a0_v, sema0)
    pltpu.async_copy(o2_hbm.at[pl.ds(base, _CW)], b0_v, semb0)

    def dchunk_body(h, _):
        c0 = 2 * h
        c1 = 2 * h + 1
        pltpu.async_copy(o1_hbm.at[pl.ds(base + c1 * _CW, _CW)], a1_v, sema1)
        pltpu.async_copy(o2_hbm.at[pl.ds(base + c1 * _CW, _CW)], b1_v, semb1)
        pltpu.make_async_copy(o1_hbm.at[pl.ds(0, _CW)], a0_v, sema0).wait()
        pltpu.make_async_copy(o2_hbm.at[pl.ds(0, _CW)], b0_v, semb0).wait()
        compute_chunk(c0, a0_v, b0_v)

        @pl.when(h < _NCH // 2 - 1)
        def _():
            c2 = 2 * h + 2
            pltpu.async_copy(o1_hbm.at[pl.ds(base + c2 * _CW, _CW)], a0_v, sema0)
            pltpu.async_copy(o2_hbm.at[pl.ds(base + c2 * _CW, _CW)], b0_v, semb0)

        pltpu.make_async_copy(o1_hbm.at[pl.ds(0, _CW)], a1_v, sema1).wait()
        pltpu.make_async_copy(o2_hbm.at[pl.ds(0, _CW)], b1_v, semb1).wait()
        compute_chunk(c1, a1_v, b1_v)
        return 0

    lax.fori_loop(0, _NCH // 2, dchunk_body, 0)

    def publish_and_sum(row, buf):
        row_i_v[...] = row
        pltpu.sync_copy(row_i_v, cnts_sh.at[pl.ds((buf * _NT + wid) * _L, _L)])
        plsc.subcore_barrier()
        pltpu.sync_copy(cnts_sh.at[pl.ds(buf * _NT * _L, _NT * _L)], all_i_v)
        tot = jnp.zeros((_L,), jnp.int32)
        for t in range(_NT):
            tot = tot + all_i_v[pl.ds(t * _L, _L)]
        return tot

    def count_rows(fmids, nb):
        accs = [jnp.zeros((_L,), jnp.int32) for _ in range(nb)]
        for j in range(_NV):
            v = vals_v[pl.ds(j * _L, _L)]
            for b in range(1, nb):
                accs[b] = accs[b] + jnp.where(v >= fmids[b], 1, 0)
        row = jnp.zeros((_L,), jnp.int32)
        for b in range(1, nb):
            row = jnp.where(iota == b, jnp.sum(accs[b]), row)
        return row

    def round_fn(r, prefix):
        shift = 27 - 4 * r
        fmids = [
            lax.bitcast_convert_type(
                prefix + jnp.left_shift(jnp.int32(b), shift), jnp.float32)
            for b in range(16)
        ]
        tot = publish_and_sum(count_rows(fmids, 16), jnp.bitwise_and(r, 1))
        nib = jnp.max(jnp.where(tot >= K, iota, 0))
        return prefix + jnp.left_shift(nib, shift)

    prefix = lax.fori_loop(0, 7, round_fn, jnp.int32(0))
    fmids = [lax.bitcast_convert_type(prefix + jnp.int32(b), jnp.float32)
             for b in range(8)]
    tot = publish_and_sum(count_rows(fmids, 8), jnp.int32(1))
    lo = prefix + jnp.max(jnp.where(tot >= K, iota, 0))

    flo = lax.bitcast_convert_type(lo, jnp.float32)
    sum_gt = jnp.zeros((_L,), jnp.float32)
    cnt_gt = jnp.zeros((_L,), jnp.int32)
    for j in range(_NV):
        v = vals_v[pl.ds(j * _L, _L)]
        gt = v > flo
        sum_gt = sum_gt + jnp.where(gt, v, 0.0)
        cnt_gt = cnt_gt + jnp.where(gt, 1, 0)
    row_f_v[...] = sum_gt
    row_i_v[...] = cnt_gt
    pltpu.sync_copy(row_f_v, sums_sh.at[pl.ds(wid * _L, _L)])
    pltpu.sync_copy(row_i_v, cnts_sh.at[pl.ds(wid * _L, _L)])
    plsc.subcore_barrier()

    @pl.when(wid == 0)
    def _():
        pltpu.sync_copy(cnts_sh.at[pl.ds(0, _NT * _L)], all_i_v)
        pltpu.sync_copy(sums_sh, all_f_v)
        tots = jnp.zeros((_L,), jnp.float32)
        totc = jnp.zeros((_L,), jnp.int32)
        for t in range(_NT):
            tots = tots + all_f_v[pl.ds(t * _L, _L)]
            totc = totc + all_i_v[pl.ds(t * _L, _L)]
        s = jnp.sum(tots)
        c = jnp.sum(totc).astype(jnp.float32)
        res = (s + (jnp.float32(K) - c) * flo) * (1.0 / K)
        res_v[...] = jnp.full((_L,), res, jnp.float32)
        pltpu.sync_copy(res_v, out_hbm)


_full_sc = functools.partial(
    pl.kernel,
    mesh=plsc.VectorSubcoreMesh(core_axis_name="c", subcore_axis_name="s",
                                num_cores=1),
    out_type=jax.ShapeDtypeStruct((_L,), jnp.float32),
    compiler_params=pltpu.CompilerParams(needs_layout_passes=False),
    scratch_types=[
        pltpu.VMEM((_CW,), jnp.float32),
        pltpu.VMEM((_CW,), jnp.float32),
        pltpu.VMEM((_CW,), jnp.float32),
        pltpu.VMEM((_CW,), jnp.float32),
        pltpu.VMEM((_PW,), jnp.int32),
        pltpu.VMEM((_PW,), jnp.float32),
        pltpu.VMEM((_L,), jnp.int32),
        pltpu.VMEM((_NT * _L,), jnp.int32),
        pltpu.VMEM((_L,), jnp.float32),
        pltpu.VMEM((_NT * _L,), jnp.float32),
        pltpu.VMEM((_L,), jnp.float32),
        pltpu.VMEM_SHARED((2 * _NT * _L,), jnp.int32),
        pltpu.VMEM_SHARED((_NT * _L,), jnp.float32),
        pltpu.SemaphoreType.DMA,
        pltpu.SemaphoreType.DMA,
        pltpu.SemaphoreType.DMA,
        pltpu.SemaphoreType.DMA,
    ],
)(_full_sc_body)


def kernel(output1, output2, label):
    out = _full_sc(output1.reshape(N * D), output2.reshape(N * D),
                   label.astype(jnp.int32))
    return out[0]


# speedup vs baseline: 1.4366x; 1.4366x over previous
import functools

import jax
import jax.numpy as jnp
from jax import lax
from jax.experimental import pallas as pl
from jax.experimental.pallas import tpu as pltpu
from jax.experimental.pallas import tpu_sc as plsc

N = 16384
D = 128
K = N // 2
MARGIN = 2.0
EPS = 1e-6

_RB = 32

_L = 16
_NT = 16
_PW = N // _NT
_NV = _PW // _L


def _dense_body(o1_ref, o2_ref, lab_ref, out_ref):
    d = o1_ref[...] - o2_ref[...] + EPS
    s = jnp.sum(d * d, axis=2)
    dist = jnp.sqrt(s)
    labf = lab_ref[...].astype(jnp.float32)
    hinge = jnp.maximum(MARGIN - dist, 0.0)
    out_ref[...] = labf * s + (1.0 - labf) * hinge * hinge


def _select_sc_body(loss_hbm, out_hbm, vals_v, row_i_v, all_i_v, row_f_v,
                    all_f_v, res_v, cnts_sh, sums_sh):
    wid = lax.axis_index("s")
    pltpu.sync_copy(loss_hbm.at[pl.ds(wid * _PW, _PW)], vals_v)

    iota = lax.iota(jnp.int32, _L)

    def publish_and_sum(row, buf):
        row_i_v[...] = row
        pltpu.sync_copy(row_i_v, cnts_sh.at[pl.ds((buf * _NT + wid) * _L, _L)])
        plsc.subcore_barrier()
        pltpu.sync_copy(cnts_sh.at[pl.ds(buf * _NT * _L, _NT * _L)], all_i_v)
        tot = jnp.zeros((_L,), jnp.int32)
        for t in range(_NT):
            tot = tot + all_i_v[pl.ds(t * _L, _L)]
        return tot

    def count_rows(fmids, nb):
        accs = [jnp.zeros((_L,), jnp.int32) for _ in range(nb)]
        for j in range(_NV):
            v = vals_v[pl.ds(j * _L, _L)]
            for b in range(1, nb):
                accs[b] = accs[b] + jnp.where(v >= fmids[b], 1, 0)
        row = jnp.zeros((_L,), jnp.int32)
        for b in range(1, nb):
            row = jnp.where(iota == b, jnp.sum(accs[b]), row)
        return row

    def round_fn(r, prefix):
        shift = 27 - 4 * r
        fmids = [
            lax.bitcast_convert_type(
                prefix + jnp.left_shift(jnp.int32(b), shift), jnp.float32)
            for b in range(16)
        ]
        tot = publish_and_sum(count_rows(fmids, 16), jnp.bitwise_and(r, 1))
        nib = jnp.max(jnp.where(tot >= K, iota, 0))
        return prefix + jnp.left_shift(nib, shift)

    prefix = lax.fori_loop(0, 7, round_fn, jnp.int32(0))
    fmids = [lax.bitcast_convert_type(prefix + jnp.int32(b), jnp.float32)
             for b in range(8)]
    tot = publish_and_sum(count_rows(fmids, 8), jnp.int32(1))
    lo = prefix + jnp.max(jnp.where(tot >= K, iota, 0))

    flo = lax.bitcast_convert_type(lo, jnp.float32)
    sum_gt = jnp.zeros((_L,), jnp.float32)
    cnt_gt = jnp.zeros((_L,), jnp.int32)
    for j in range(_NV):
        v = vals_v[pl.ds(j * _L, _L)]
        gt = v > flo
        sum_gt = sum_gt + jnp.where(gt, v, 0.0)
        cnt_gt = cnt_gt + jnp.where(gt, 1, 0)
    row_f_v[...] = sum_gt
    row_i_v[...] = cnt_gt
    pltpu.sync_copy(row_f_v, sums_sh.at[pl.ds(wid * _L, _L)])
    pltpu.sync_copy(row_i_v, cnts_sh.at[pl.ds(wid * _L, _L)])
    plsc.subcore_barrier()

    @pl.when(wid == 0)
    def _():
        pltpu.sync_copy(cnts_sh.at[pl.ds(0, _NT * _L)], all_i_v)
        pltpu.sync_copy(sums_sh, all_f_v)
        tots = jnp.zeros((_L,), jnp.float32)
        totc = jnp.zeros((_L,), jnp.int32)
        for t in range(_NT):
            tots = tots + all_f_v[pl.ds(t * _L, _L)]
            totc = totc + all_i_v[pl.ds(t * _L, _L)]
        s = jnp.sum(tots)
        c = jnp.sum(totc).astype(jnp.float32)
        res = (s + (jnp.float32(K) - c) * flo) * (1.0 / K)
        res_v[...] = jnp.full((_L,), res, jnp.float32)
        pltpu.sync_copy(res_v, out_hbm)


_select_sc = functools.partial(
    pl.kernel,
    mesh=plsc.VectorSubcoreMesh(core_axis_name="c", subcore_axis_name="s",
                                num_cores=1),
    out_type=jax.ShapeDtypeStruct((_L,), jnp.float32),
    compiler_params=pltpu.CompilerParams(needs_layout_passes=False,
                                         skip_device_barrier=True),
    scratch_types=[
        pltpu.VMEM((_PW,), jnp.float32),
        pltpu.VMEM((_L,), jnp.int32),
        pltpu.VMEM((_NT * _L,), jnp.int32),
        pltpu.VMEM((_L,), jnp.float32),
        pltpu.VMEM((_NT * _L,), jnp.float32),
        pltpu.VMEM((_L,), jnp.float32),
        pltpu.VMEM_SHARED((2 * _NT * _L,), jnp.int32),
        pltpu.VMEM_SHARED((_NT * _L,), jnp.float32),
    ],
)(_select_sc_body)


def kernel(output1, output2, label):
    o1 = output1.reshape(N // D, D, D)
    o2 = output2.reshape(N // D, D, D)
    lab = label.astype(jnp.int32).reshape(N // D, D)

    grid = (N // D) // _RB
    loss = pl.pallas_call(
        _dense_body,
        grid=(grid,),
        in_specs=[
            pl.BlockSpec((_RB, D, D), lambda i: (i, 0, 0)),
            pl.BlockSpec((_RB, D, D), lambda i: (i, 0, 0)),
            pl.BlockSpec((_RB, D), lambda i: (i, 0)),
        ],
        out_specs=pl.BlockSpec((_RB, D), lambda i: (i, 0)),
        out_shape=jax.ShapeDtypeStruct((N // D, D), jnp.float32),
    )(o1, o2, lab)

    out = _select_sc(loss.reshape(N))
    return out[0]


# speedup vs baseline: 1.4444x; 1.0054x over previous
import functools

import jax
import jax.numpy as jnp
from jax import lax
from jax.experimental import pallas as pl
from jax.experimental.pallas import tpu as pltpu
from jax.experimental.pallas import tpu_sc as plsc

N = 16384
D = 128
K = N // 2
MARGIN = 2.0
EPS = 1e-6

_RB = 64

_L = 16
_NT = 16
_PW = N // _NT
_NV = _PW // _L


def _dense_body(o1_ref, o2_ref, lab_ref, out_ref):
    d = o1_ref[...] - o2_ref[...] + EPS
    s = jnp.sum(d * d, axis=2)
    dist = jnp.sqrt(s)
    labf = lab_ref[...].astype(jnp.float32)
    hinge = jnp.maximum(MARGIN - dist, 0.0)
    out_ref[...] = labf * s + (1.0 - labf) * hinge * hinge


def _select_sc_body(loss_hbm, out_hbm, vals_v, row_i_v, all_i_v, row_f_v,
                    all_f_v, res_v, cnts_sh, sums_sh):
    wid = lax.axis_index("s")
    pltpu.sync_copy(loss_hbm.at[pl.ds(wid * _PW, _PW)], vals_v)

    iota = lax.iota(jnp.int32, _L)

    def publish_and_sum(row, buf):
        row_i_v[...] = row
        pltpu.sync_copy(row_i_v, cnts_sh.at[pl.ds((buf * _NT + wid) * _L, _L)])
        plsc.subcore_barrier()
        pltpu.sync_copy(cnts_sh.at[pl.ds(buf * _NT * _L, _NT * _L)], all_i_v)
        tot = jnp.zeros((_L,), jnp.int32)
        for t in range(_NT):
            tot = tot + all_i_v[pl.ds(t * _L, _L)]
        return tot

    def count_rows(fmids, nb):
        accs = [jnp.zeros((_L,), jnp.int32) for _ in range(nb)]
        for j in range(_NV):
            v = vals_v[pl.ds(j * _L, _L)]
            for b in range(1, nb):
                accs[b] = accs[b] + jnp.where(v >= fmids[b], 1, 0)
        row = jnp.zeros((_L,), jnp.int32)
        for b in range(1, nb):
            row = jnp.where(iota == b, jnp.sum(accs[b]), row)
        return row

    def round_fn(r, prefix):
        shift = 27 - 4 * r
        fmids = [
            lax.bitcast_convert_type(
                prefix + jnp.left_shift(jnp.int32(b), shift), jnp.float32)
            for b in range(16)
        ]
        tot = publish_and_sum(count_rows(fmids, 16), jnp.bitwise_and(r, 1))
        nib = jnp.max(jnp.where(tot >= K, iota, 0))
        return prefix + jnp.left_shift(nib, shift)

    prefix = lax.fori_loop(0, 7, round_fn, jnp.int32(0))
    fmids = [lax.bitcast_convert_type(prefix + jnp.int32(b), jnp.float32)
             for b in range(8)]
    tot = publish_and_sum(count_rows(fmids, 8), jnp.int32(1))
    lo = prefix + jnp.max(jnp.where(tot >= K, iota, 0))

    flo = lax.bitcast_convert_type(lo, jnp.float32)
    sum_gt = jnp.zeros((_L,), jnp.float32)
    cnt_gt = jnp.zeros((_L,), jnp.int32)
    for j in range(_NV):
        v = vals_v[pl.ds(j * _L, _L)]
        gt = v > flo
        sum_gt = sum_gt + jnp.where(gt, v, 0.0)
        cnt_gt = cnt_gt + jnp.where(gt, 1, 0)
    row_f_v[...] = sum_gt
    row_i_v[...] = cnt_gt
    pltpu.sync_copy(row_f_v, sums_sh.at[pl.ds(wid * _L, _L)])
    pltpu.sync_copy(row_i_v, cnts_sh.at[pl.ds(wid * _L, _L)])
    plsc.subcore_barrier()

    @pl.when(wid == 0)
    def _():
        pltpu.sync_copy(cnts_sh.at[pl.ds(0, _NT * _L)], all_i_v)
        pltpu.sync_copy(sums_sh, all_f_v)
        tots = jnp.zeros((_L,), jnp.float32)
        totc = jnp.zeros((_L,), jnp.int32)
        for t in range(_NT):
            tots = tots + all_f_v[pl.ds(t * _L, _L)]
            totc = totc + all_i_v[pl.ds(t * _L, _L)]
        s = jnp.sum(tots)
        c = jnp.sum(totc).astype(jnp.float32)
        res = (s + (jnp.float32(K) - c) * flo) * (1.0 / K)
        res_v[...] = jnp.full((_L,), res, jnp.float32)
        pltpu.sync_copy(res_v, out_hbm)


_select_sc = functools.partial(
    pl.kernel,
    mesh=plsc.VectorSubcoreMesh(core_axis_name="c", subcore_axis_name="s",
                                num_cores=1),
    out_type=jax.ShapeDtypeStruct((_L,), jnp.float32),
    compiler_params=pltpu.CompilerParams(needs_layout_passes=False,
                                         skip_device_barrier=True),
    scratch_types=[
        pltpu.VMEM((_PW,), jnp.float32),
        pltpu.VMEM((_L,), jnp.int32),
        pltpu.VMEM((_NT * _L,), jnp.int32),
        pltpu.VMEM((_L,), jnp.float32),
        pltpu.VMEM((_NT * _L,), jnp.float32),
        pltpu.VMEM((_L,), jnp.float32),
        pltpu.VMEM_SHARED((2 * _NT * _L,), jnp.int32),
        pltpu.VMEM_SHARED((_NT * _L,), jnp.float32),
    ],
)(_select_sc_body)


def kernel(output1, output2, label):
    o1 = output1.reshape(N // D, D, D)
    o2 = output2.reshape(N // D, D, D)
    lab = label.astype(jnp.int32).reshape(N // D, D)

    grid = (N // D) // _RB
    loss = pl.pallas_call(
        _dense_body,
        grid=(grid,),
        in_specs=[
            pl.BlockSpec((_RB, D, D), lambda i: (i, 0, 0)),
            pl.BlockSpec((_RB, D, D), lambda i: (i, 0, 0)),
            pl.BlockSpec((_RB, D), lambda i: (i, 0)),
        ],
        out_specs=pl.BlockSpec((_RB, D), lambda i: (i, 0)),
        out_shape=jax.ShapeDtypeStruct((N // D, D), jnp.float32),
    )(o1, o2, lab)

    out = _select_sc(loss.reshape(N))
    return out[0]
